# baseline (speedup 1.0000x reference)
import functools

import jax
import jax.numpy as jnp
from jax import lax
from jax.experimental import pallas as pl
from jax.experimental.pallas import tpu as pltpu
from jax.experimental.pallas import tpu_sc as plsc

B = 16384
N = 100000
D = 64
L = 16

_info = plsc.get_sparse_core_info()
NC, NS = _info.num_cores, _info.num_subcores
NW = NC * NS
DPW = D // NW
BPW = B // NW
SEL_CH = 2048


def _phase_a(i_hbm, j_hbm, k_hbm, wt_hbm, vt_hbm, ut_hbm, sel_hbm,
             col_v, idx_v, stage_v, sem):
    wid = lax.axis_index("s") * NC + lax.axis_index("c")

    for t, (tab, idx_hbm) in enumerate(((wt_hbm, i_hbm), (vt_hbm, j_hbm),
                                        (ut_hbm, k_hbm))):
        pltpu.sync_copy(idx_hbm, idx_v)
        for dd in range(DPW):
            d = wid * DPW + dd
            pltpu.sync_copy(tab.at[d], col_v)
            for ch in range(B // SEL_CH):

                def blk(g, carry):
                    iv = idx_v[pl.ds(ch * SEL_CH + g * L, L)]
                    stage_v[pl.ds(g * L, L)] = plsc.load_gather(col_v, [iv])
                    return carry

                lax.fori_loop(0, SEL_CH // L, blk, 0)
                pltpu.sync_copy(
                    stage_v,
                    sel_hbm.at[t * D + d, pl.ds(ch * SEL_CH, SEL_CH)])


def _phase_b(sel_hbm, out_hbm, sel_v, out_v, sem):
    wid = lax.axis_index("s") * NC + lax.axis_index("c")
    base = wid * BPW

    def ld(p, carry):
        pltpu.async_copy(sel_hbm.at[p, pl.ds(base, BPW)], sel_v.at[p], sem)
        return carry

    lax.fori_loop(0, 3 * D, ld, 0)

    def drain(p, carry):
        pltpu.make_async_copy(sel_hbm.at[0, pl.ds(0, BPW)], sel_v.at[p],
                              sem).wait()
        return carry

    lax.fori_loop(0, 3 * D, drain, 0)

    def blk(g, carry):
        sl = pl.ds(g * L, L)
        acc = jnp.zeros((L,), jnp.float32)
        for d in range(D):
            acc = acc + sel_v[d, sl] * sel_v[D + d, sl] * sel_v[2 * D + d, sl]
        out_v[sl] = 1.0 / (1.0 + jnp.exp(-acc))
        return carry

    lax.fori_loop(0, BPW // L, blk, 0)

    pltpu.sync_copy(out_v, out_hbm.at[pl.ds(base, BPW)])


@functools.partial(jax.jit, static_argnums=())
def kernel(indices, W, V, U):
    idx = indices.astype(jnp.int32)
    i_idx, j_idx, k_idx = idx[:, 0], idx[:, 1], idx[:, 2]
    WT, VT, UT = W.T, V.T, U.T

    mesh = plsc.VectorSubcoreMesh(core_axis_name="c", subcore_axis_name="s")
    cp = pltpu.CompilerParams(needs_layout_passes=False,
                              use_tc_tiling_on_sc=False)

    sel = pl.kernel(
        _phase_a,
        mesh=mesh,
        out_type=jax.ShapeDtypeStruct((3 * D, B), jnp.float32),
        scratch_types=[
            pltpu.VMEM((N,), jnp.float32),
            pltpu.VMEM((B,), jnp.int32),
            pltpu.VMEM((SEL_CH,), jnp.float32),
            pltpu.SemaphoreType.DMA,
        ],
        compiler_params=cp,
    )(i_idx, j_idx, k_idx, WT, VT, UT)

    return pl.kernel(
        _phase_b,
        mesh=mesh,
        out_type=jax.ShapeDtypeStruct((B,), jnp.float32),
        scratch_types=[
            pltpu.VMEM((3 * D, BPW), jnp.float32),
            pltpu.VMEM((BPW,), jnp.float32),
            pltpu.SemaphoreType.DMA,
        ],
        compiler_params=cp,
    )(sel)


# speedup vs baseline: 1.0756x; 1.0756x over previous
import functools

import jax
import jax.numpy as jnp
from jax import lax
from jax.experimental import pallas as pl
from jax.experimental.pallas import tpu as pltpu
from jax.experimental.pallas import tpu_sc as plsc

B = 16384
N = 100000
D = 64
L = 16

_info = plsc.get_sparse_core_info()
NC, NS = _info.num_cores, _info.num_subcores
NW = NC * NS
DPW = D // NW
BPW = B // NW
SEL_CH = 2048


def _phase_a(i_hbm, j_hbm, k_hbm, wt_hbm, vt_hbm, ut_hbm, sel_hbm,
             col_v, idx_v, stage_a, stage_b, sem):
    wid = lax.axis_index("s") * NC + lax.axis_index("c")

    stages = (stage_a, stage_b)
    nch = B // SEL_CH
    UNROLL = 4
    k = 0

    for t, (tab, idx_hbm) in enumerate(((wt_hbm, i_hbm), (vt_hbm, j_hbm),
                                        (ut_hbm, k_hbm))):
        pltpu.sync_copy(idx_hbm, idx_v)
        for dd in range(DPW):
            d = wid * DPW + dd
            pltpu.sync_copy(tab.at[d], col_v)
            for ch in range(nch):
                sb = stages[k % 2]
                if k >= 2:
                    pltpu.make_async_copy(
                        stage_a, sel_hbm.at[0, pl.ds(0, SEL_CH)], sem).wait()

                def blk(g, carry, ch=ch, sb=sb):
                    for u in range(UNROLL):
                        o = (g * UNROLL + u) * L
                        iv = idx_v[pl.ds(ch * SEL_CH + o, L)]
                        sb[pl.ds(o, L)] = plsc.load_gather(col_v, [iv])
                    return carry

                lax.fori_loop(0, SEL_CH // L // UNROLL, blk, 0)
                pltpu.async_copy(
                    sb, sel_hbm.at[t * D + d, pl.ds(ch * SEL_CH, SEL_CH)],
                    sem)
                k += 1

    for _ in range(2):
        pltpu.make_async_copy(stage_a, sel_hbm.at[0, pl.ds(0, SEL_CH)],
                              sem).wait()


def _phase_b(sel_hbm, out_hbm, sel_v, out_v, sem):
    wid = lax.axis_index("s") * NC + lax.axis_index("c")
    base = wid * BPW

    def ld(p, carry):
        pltpu.async_copy(sel_hbm.at[p, pl.ds(base, BPW)], sel_v.at[p], sem)
        return carry

    lax.fori_loop(0, 3 * D, ld, 0)

    def drain(p, carry):
        pltpu.make_async_copy(sel_hbm.at[0, pl.ds(0, BPW)], sel_v.at[p],
                              sem).wait()
        return carry

    lax.fori_loop(0, 3 * D, drain, 0)

    def blk(g, carry):
        sl = pl.ds(g * L, L)
        acc = jnp.zeros((L,), jnp.float32)
        for d in range(D):
            acc = acc + sel_v[d, sl] * sel_v[D + d, sl] * sel_v[2 * D + d, sl]
        out_v[sl] = 1.0 / (1.0 + jnp.exp(-acc))
        return carry

    lax.fori_loop(0, BPW // L, blk, 0)

    pltpu.sync_copy(out_v, out_hbm.at[pl.ds(base, BPW)])


@functools.partial(jax.jit, static_argnums=())
def kernel(indices, W, V, U):
    idx = indices.astype(jnp.int32)
    i_idx, j_idx, k_idx = idx[:, 0], idx[:, 1], idx[:, 2]
    WT, VT, UT = W.T, V.T, U.T

    mesh = plsc.VectorSubcoreMesh(core_axis_name="c", subcore_axis_name="s")
    cp = pltpu.CompilerParams(needs_layout_passes=False,
                              use_tc_tiling_on_sc=False)

    sel = pl.kernel(
        _phase_a,
        mesh=mesh,
        out_type=jax.ShapeDtypeStruct((3 * D, B), jnp.float32),
        scratch_types=[
            pltpu.VMEM((N,), jnp.float32),
            pltpu.VMEM((B,), jnp.int32),
            pltpu.VMEM((SEL_CH,), jnp.float32),
            pltpu.VMEM((SEL_CH,), jnp.float32),
            pltpu.SemaphoreType.DMA,
        ],
        compiler_params=cp,
    )(i_idx, j_idx, k_idx, WT, VT, UT)

    return pl.kernel(
        _phase_b,
        mesh=mesh,
        out_type=jax.ShapeDtypeStruct((B,), jnp.float32),
        scratch_types=[
            pltpu.VMEM((3 * D, BPW), jnp.float32),
            pltpu.VMEM((BPW,), jnp.float32),
            pltpu.SemaphoreType.DMA,
        ],
        compiler_params=cp,
    )(sel)
